# baseline (speedup 1.0000x reference)
import jax
import jax.numpy as jnp
from jax.experimental import pallas as pl
from jax.experimental.pallas import tpu as pltpu

_N = 1024


def _dot(a, b):
    return jax.lax.dot_general(
        a, b, (((1,), (0,)), ((), ())), preferred_element_type=jnp.float32
    )


def _adot(adj, us):
    if len(us) == 1:
        return [_dot(adj, us[0])]
    d = us[0].shape[1]
    r = _dot(adj, jnp.concatenate(us, axis=1))
    return [r[:, i * d:(i + 1) * d] for i in range(len(us))]


def _mpl(adj, hs, w):
    return [jnp.maximum(t, 0.0)
            for t in _adot(adj, [_dot(h, w) for h in hs])]


def _bn(h, g, b, o, eps=1e-5):
    del o
    m = jnp.mean(h, axis=0, keepdims=True)
    c = h - m
    v = jnp.mean(c * c, axis=0, keepdims=True)
    return c * jax.lax.rsqrt(v + eps) * g + b


def _fwd_kernel(
    x_ref, adj_ref, wfst_ref, g11_ref, g12_ref, g21_ref, g22_ref,
    w3_ref, w4_ref, w5_ref, bn1g_ref, bn1b_ref, g31_ref, g32_ref,
    bn2g_ref, bn2b_ref, g41_ref, g42_ref, wt_ref, wl_ref,
    bn3g_ref, bn3b_ref, fw1_ref, fb1_ref, fw2_ref, fb2_ref,
    wfin_ref, out_ref,
):
    adj = adj_ref[...]
    o = jnp.full((1, _N), 1.0 / _N, dtype=jnp.float32)
    nb = x_ref.shape[0]
    rng = range(nb)

    skip1 = [x_ref[i] * wfst_ref[...] for i in rng]

    h = _mpl(adj, skip1, g11_ref[...])
    h = _mpl(adj, h, g12_ref[...])
    g1 = [jnp.concatenate([skip1[i], h[i]], axis=1) for i in rng]

    h = _mpl(adj, g1, g21_ref[...])
    h = _mpl(adj, h, g22_ref[...])
    g2 = [jnp.concatenate([g1[i], h[i]], axis=1) for i in rng]

    t = _mpl(adj, g2, w3_ref[...])
    t = _mpl(adj, t, w4_ref[...])

    s2 = [_dot(skip1[i], w5_ref[0:64, :]) + _dot(t[i], w5_ref[64:320, :])
          for i in rng]
    skip2 = [_bn(s2[i], bn1g_ref[...], bn1b_ref[...], o) for i in rng]

    h = _mpl(adj, skip2, g31_ref[...])
    h = _mpl(adj, h, g32_ref[...])
    g3 = [jnp.concatenate([skip2[i], h[i]], axis=1) for i in rng]
    s3a = [_bn(g3[i], bn2g_ref[...], bn2b_ref[...], o) for i in rng]

    h = _mpl(adj, s3a, g41_ref[...])
    h2 = _mpl(adj, h, g42_ref[...])
    u = [_dot(s3a[i], wt_ref[0:512, :]) + _dot(h2[i], wt_ref[512:1024, :])
         for i in rng]
    t10 = [jnp.maximum(t, 0.0) for t in _adot(adj, u)]

    s3 = [_dot(t10[i], wl_ref[0:1024, :]) + _dot(skip2[i], wl_ref[1024:1280, :])
          for i in rng]
    skip3 = [_bn(s3[i], bn3g_ref[...], bn3b_ref[...], o) for i in rng]

    last = [
        jnp.maximum(
            _dot(skip3[i], fw1_ref[0:256, :])
            + _dot(skip2[i], fw1_ref[256:512, :])
            + _dot(skip1[i], fw1_ref[512:576, :])
            + fb1_ref[...],
            0.0,
        )
        for i in rng
    ]

    v = [_dot(last[i], fw2_ref[...]) + fb2_ref[...] for i in rng]
    v = [v[i] * wfin_ref[0, 0] for i in rng]

    for i in rng:
        e = jnp.exp(v[i] - jnp.max(v[i], axis=0, keepdims=True))
        out_ref[i] = e / jnp.sum(e, axis=0, keepdims=True)


def kernel(x, adj_mat, edge_index, batch_size, W_fst, gb1_w1, gb1_w2,
           gb2_w1, gb2_w2, W_thrd_, W_frth, W_ffth, bn1_g, bn1_b,
           gb3_w1, gb3_w2, bn2_g, bn2_b, gb4_w1, gb4_w2, W_tnth,
           W_lvnth, bn3_g, bn3_b, ff_w1, ff_b1, ff_w2, ff_b2, W_final):
    del edge_index, batch_size
    bsz = x.shape[0]
    bpp = 2
    assert bsz % bpp == 0

    def const2(a):
        return pl.BlockSpec(a.shape, lambda b: (0,) * a.ndim)

    operands = (
        x, adj_mat, W_fst, gb1_w1, gb1_w2, gb2_w1, gb2_w2,
        W_thrd_, W_frth, W_ffth,
        bn1_g.reshape(1, -1), bn1_b.reshape(1, -1),
        gb3_w1, gb3_w2,
        bn2_g.reshape(1, -1), bn2_b.reshape(1, -1),
        gb4_w1, gb4_w2, W_tnth, W_lvnth,
        bn3_g.reshape(1, -1), bn3_b.reshape(1, -1),
        ff_w1, ff_b1.reshape(1, -1), ff_w2, ff_b2.reshape(1, -1), W_final,
    )
    in_specs = [pl.BlockSpec((bpp, _N, 1), lambda b: (b, 0, 0))]
    in_specs += [const2(a) for a in operands[1:]]

    out = pl.pallas_call(
        _fwd_kernel,
        grid=(bsz // bpp,),
        in_specs=in_specs,
        out_specs=pl.BlockSpec((bpp, _N, 1), lambda b: (b, 0, 0)),
        out_shape=jax.ShapeDtypeStruct((bsz, _N, 1), jnp.float32),
        compiler_params=pltpu.CompilerParams(
            dimension_semantics=("parallel",),
            vmem_limit_bytes=120 * 1024 * 1024,
        ),
    )(*operands)
    return out.reshape(bsz, _N)


# speedup vs baseline: 1.0208x; 1.0208x over previous
import jax
import jax.numpy as jnp
from jax.experimental import pallas as pl
from jax.experimental.pallas import tpu as pltpu

_N = 1024


def _dot(a, b):
    return jax.lax.dot_general(
        a, b, (((1,), (0,)), ((), ())), preferred_element_type=jnp.float32
    )


def _adot(adj, us):
    return [_dot(adj, u) for u in us]


def _mpl(adj, hs, w):
    return [jnp.maximum(t, 0.0)
            for t in _adot(adj, [_dot(h, w) for h in hs])]


def _bn(h, g, b, o, eps=1e-5):
    del o
    m = jnp.mean(h, axis=0, keepdims=True)
    q = jnp.mean(h * h, axis=0, keepdims=True)
    v = q - m * m
    return (h - m) * jax.lax.rsqrt(v + eps) * g + b


def _fwd_kernel(
    x_ref, adj_ref, wfst_ref, g11_ref, g12_ref, g21_ref, g22_ref,
    w3_ref, w4_ref, w5_ref, bn1g_ref, bn1b_ref, g31_ref, g32_ref,
    bn2g_ref, bn2b_ref, g41_ref, g42_ref, wt_ref, wl_ref,
    bn3g_ref, bn3b_ref, fw1_ref, fb1_ref, fw2_ref, fb2_ref,
    wfin_ref, out_ref,
):
    adj = adj_ref[...]
    o = jnp.full((1, _N), 1.0 / _N, dtype=jnp.float32)
    nb = x_ref.shape[0]
    rng = range(nb)

    skip1 = [x_ref[i] * wfst_ref[...] for i in rng]

    h = _mpl(adj, skip1, g11_ref[...])
    h = _mpl(adj, h, g12_ref[...])
    g1 = [jnp.concatenate([skip1[i], h[i]], axis=1) for i in rng]

    h = _mpl(adj, g1, g21_ref[...])
    h = _mpl(adj, h, g22_ref[...])
    g2 = [jnp.concatenate([g1[i], h[i]], axis=1) for i in rng]

    t = _mpl(adj, g2, w3_ref[...])
    t = _mpl(adj, t, w4_ref[...])

    s2 = [_dot(skip1[i], w5_ref[0:64, :]) + _dot(t[i], w5_ref[64:320, :])
          for i in rng]
    skip2 = [_bn(s2[i], bn1g_ref[...], bn1b_ref[...], o) for i in rng]

    h = _mpl(adj, skip2, g31_ref[...])
    h = _mpl(adj, h, g32_ref[...])
    g3 = [jnp.concatenate([skip2[i], h[i]], axis=1) for i in rng]
    s3a = [_bn(g3[i], bn2g_ref[...], bn2b_ref[...], o) for i in rng]

    h = _mpl(adj, s3a, g41_ref[...])
    h2 = _mpl(adj, h, g42_ref[...])
    u = [_dot(s3a[i], wt_ref[0:512, :]) + _dot(h2[i], wt_ref[512:1024, :])
         for i in rng]
    t10 = [jnp.maximum(t, 0.0) for t in _adot(adj, u)]

    s3 = [_dot(t10[i], wl_ref[0:1024, :]) + _dot(skip2[i], wl_ref[1024:1280, :])
          for i in rng]
    skip3 = [_bn(s3[i], bn3g_ref[...], bn3b_ref[...], o) for i in rng]

    last = [
        jnp.maximum(
            _dot(skip3[i], fw1_ref[0:256, :])
            + _dot(skip2[i], fw1_ref[256:512, :])
            + _dot(skip1[i], fw1_ref[512:576, :])
            + fb1_ref[...],
            0.0,
        )
        for i in rng
    ]

    v = [_dot(last[i], fw2_ref[...]) + fb2_ref[...] for i in rng]
    v = [v[i] * wfin_ref[0, 0] for i in rng]

    for i in rng:
        e = jnp.exp(v[i] - jnp.max(v[i], axis=0, keepdims=True))
        out_ref[i] = e / jnp.sum(e, axis=0, keepdims=True)


def kernel(x, adj_mat, edge_index, batch_size, W_fst, gb1_w1, gb1_w2,
           gb2_w1, gb2_w2, W_thrd_, W_frth, W_ffth, bn1_g, bn1_b,
           gb3_w1, gb3_w2, bn2_g, bn2_b, gb4_w1, gb4_w2, W_tnth,
           W_lvnth, bn3_g, bn3_b, ff_w1, ff_b1, ff_w2, ff_b2, W_final):
    del edge_index, batch_size
    bsz = x.shape[0]
    bpp = 2
    assert bsz % bpp == 0

    def const2(a):
        return pl.BlockSpec(a.shape, lambda b: (0,) * a.ndim)

    operands = (
        x, adj_mat, W_fst, gb1_w1, gb1_w2, gb2_w1, gb2_w2,
        W_thrd_, W_frth, W_ffth,
        bn1_g.reshape(1, -1), bn1_b.reshape(1, -1),
        gb3_w1, gb3_w2,
        bn2_g.reshape(1, -1), bn2_b.reshape(1, -1),
        gb4_w1, gb4_w2, W_tnth, W_lvnth,
        bn3_g.reshape(1, -1), bn3_b.reshape(1, -1),
        ff_w1, ff_b1.reshape(1, -1), ff_w2, ff_b2.reshape(1, -1), W_final,
    )
    in_specs = [pl.BlockSpec((bpp, _N, 1), lambda b: (b, 0, 0))]
    in_specs += [const2(a) for a in operands[1:]]

    out = pl.pallas_call(
        _fwd_kernel,
        grid=(bsz // bpp,),
        in_specs=in_specs,
        out_specs=pl.BlockSpec((bpp, _N, 1), lambda b: (b, 0, 0)),
        out_shape=jax.ShapeDtypeStruct((bsz, _N, 1), jnp.float32),
        compiler_params=pltpu.CompilerParams(
            dimension_semantics=("parallel",),
            vmem_limit_bytes=120 * 1024 * 1024,
        ),
    )(*operands)
    return out.reshape(bsz, _N)
